# baseline (speedup 1.0000x reference)
import functools

import jax
import jax.numpy as jnp
from jax import lax
from jax.experimental import pallas as pl
from jax.experimental.pallas import tpu as pltpu
from jax.experimental.pallas import tpu_sc as plsc

_NC = 2
_NS = 16
_LN = 16
_D = 128
_DSUB = _D // _LN
_KPC = _DSUB // _NC
_K = 512


def _round_up(x, m):
    return (x + m - 1) // m * m


def _elu(x):
    return jnp.where(x > 0, x, jnp.exp(jnp.minimum(x, 0.0)) - 1.0)


_C = 12
_NU = 3
_UR = 128
_GU = _NU * _UR
_BS = 2048
_CAP = 2 * _BS + _GU + 16
_RP = 16384


@functools.cache
def _make_spmm3(n_dst, e_pads):
    ets = [e // _NS for e in e_pads]
    nblks = [et // _BS for et in ets]
    assert all(nb * _BS == et for nb, et in zip(nblks, ets))
    R = _round_up(-(-n_dst // _C), 8)
    assert R < _RP
    acc_rows = _round_up(R + 1, 128)
    stripe = acc_rows // _NS
    trash = R
    dtail = R - (_NS - 1) * stripe
    assert 0 < dtail <= stripe and dtail % 8 == 0
    mesh = plsc.VectorSubcoreMesh(core_axis_name="c", subcore_axis_name="s",
                                  num_cores=_NC, num_subcores=_NS)
    out = jax.ShapeDtypeStruct((_C * R, _D), jnp.float32)

    scratch = [
        pltpu.VMEM((_BS,), jnp.int32),
        pltpu.VMEM((_BS,), jnp.int32),
        pltpu.VMEM((_BS,), jnp.int32),
        pltpu.VMEM((_BS,), jnp.int32),
        pltpu.VMEM((_CAP,), jnp.int32),
    ]
    scratch += [pltpu.VMEM((_UR,), jnp.int32) for _ in range(2 * _NU)]
    scratch += [pltpu.VMEM((_UR, _D), jnp.float32) for _ in range(_NU)]
    scratch += [pltpu.VMEM_SHARED((acc_rows, _D), jnp.float32)]
    scratch += [pltpu.SemaphoreType.DMA for _ in range(2 + _NU)]

    @functools.partial(
        pl.kernel,
        out_type=(out, out, out),
        mesh=mesh,
        compiler_params=pltpu.CompilerParams(needs_layout_passes=False),
        scratch_types=scratch,
    )
    def spmm3(ya, sa, da, yb, sb, db, yc, sc_, dc, z_h, outa, outb, outc,
              s0, d0, s1, d1, cbuf, *rest):
        gb = rest[:_NU]
        ib = rest[_NU:2 * _NU]
        rb = rest[2 * _NU:3 * _NU]
        acc = rest[3 * _NU]
        semA, semB = rest[3 * _NU + 1], rest[3 * _NU + 2]
        gsem = rest[3 * _NU + 3:]
        c = lax.axis_index("c")
        s = lax.axis_index("s")
        ti = jnp.full((_LN,), trash, jnp.int32)
        ones = jnp.ones((_LN,), jnp.int32)

        r0 = s * stripe
        stage = ((s0, d0, semA), (s1, d1, semB))
        sets = ((ya, sa, da, outa, 0), (yb, sb, db, outb, 1),
                (yc, sc_, dc, outc, 2))
        for y_h, src_h, dst_h, out_h, si in sets:
            et = ets[si]
            nblk = nblks[si]
            e0 = s * et

            def do_chunk(cc, _, y_h=y_h, src_h=src_h, dst_h=dst_h,
                         out_h=out_h, et=et, nblk=nblk, e0=e0):
                ch = c * (_C // _NC) + cc
                lo = ch * R
                hi = lo + R

                with jax.named_scope("ph_zero"):
                    pltpu.sync_copy(z_h, acc.at[pl.ds(r0, stripe)])
                    plsc.subcore_barrier()

                def flushg(g, _):
                    base = g * _GU
                    cps = []
                    for u in range(_NU):
                        def fill(j, _, u=u):
                            cv = cbuf[pl.ds(base + u * _UR + j * _LN, _LN)]
                            gb[u][pl.ds(j * _LN, _LN)] = cv >> 14
                            ib[u][pl.ds(j * _LN, _LN)] = cv & (_RP - 1)
                            return 0
                        lax.fori_loop(0, _UR // _LN, fill, 0)
                        cp = pltpu.make_async_copy(y_h.at[gb[u]], rb[u],
                                                   gsem[u])
                        cp.start()
                        cps.append(cp)
                    for u in range(_NU):
                        cps[u].wait()
                        pltpu.sync_copy(rb[u], acc.at[ib[u]], add=True)
                    return 0

                def scan_vregs(buf_s, buf_d, n):
                    def scan_v(i, n):
                        d = buf_d[pl.ds(i * _LN, _LN)]
                        sv = buf_s[pl.ds(i * _LN, _LN)]
                        m = (d >= lo) & (d < hi)
                        pos = plsc.cumsum(ones, mask=m) + (n - 1)
                        plsc.store_scatter(cbuf, [pos],
                                           sv * _RP + (d - lo), mask=m)
                        return n + plsc.all_reduce_population_count(m)[0]
                    return lax.fori_loop(0, _BS // _LN, scan_v, n)

                def flush_compact(n):
                    ng = n // _GU
                    lax.fori_loop(0, ng, flushg, 0)
                    done = ng * _GU

                    def comp(j, _):
                        cbuf[pl.ds(j * _LN, _LN)] = cbuf[
                            pl.ds(done + j * _LN, _LN)]
                        return 0
                    lax.fori_loop(0, _GU // _LN, comp, 0)
                    return n - done

                def pair(p, n):
                    eb0 = e0 + (2 * p) * _BS
                    eb1 = eb0 + _BS
                    c0 = pltpu.make_async_copy(src_h.at[pl.ds(eb0, _BS)],
                                               s0, semA)
                    c1 = pltpu.make_async_copy(dst_h.at[pl.ds(eb0, _BS)],
                                               d0, semA)
                    c2 = pltpu.make_async_copy(src_h.at[pl.ds(eb1, _BS)],
                                               s1, semB)
                    c3 = pltpu.make_async_copy(dst_h.at[pl.ds(eb1, _BS)],
                                               d1, semB)
                    c0.start()
                    c1.start()
                    c2.start()
                    c3.start()
                    with jax.named_scope("ph_scan"):
                        c0.wait()
                        c1.wait()
                        n = scan_vregs(s0, d0, n)
                        c2.wait()
                        c3.wait()
                        n = scan_vregs(s1, d1, n)
                    with jax.named_scope("ph_flush"):
                        n = flush_compact(n)
                    return n
                n = lax.fori_loop(0, nblk // 2, pair, 0)

                with jax.named_scope("ph_tail"):
                    def padf(j, _):
                        cbuf[pl.ds(n + j * _LN, _LN)] = ti
                        return 0
                    lax.fori_loop(0, _GU // _LN, padf, 0)
                    lax.fori_loop(0, (n + _GU - 1) // _GU, flushg, 0)
                    plsc.subcore_barrier()

                jax.named_scope("ph_drain").__enter__() if False else None
                @pl.when(s < _NS - 1)
                def _():
                    pltpu.sync_copy(acc.at[pl.ds(r0, stripe)],
                                    out_h.at[pl.ds(lo + r0, stripe)])

                @pl.when(s == _NS - 1)
                def _():
                    pltpu.sync_copy(acc.at[pl.ds(r0, dtail)],
                                    out_h.at[pl.ds(lo + r0, dtail)])
                plsc.subcore_barrier()
                return 0
            lax.fori_loop(0, _C // _NC, do_chunk, 0)

    return spmm3


def _pad_edges(idx, e_pad):
    pad = e_pad - idx.shape[1]
    dst_p = jnp.concatenate([idx[0], jnp.full((pad,), 1 << 30, jnp.int32)])
    src_p = jnp.concatenate([idx[1], jnp.zeros((pad,), jnp.int32)])
    return src_p, dst_p


def _spmm3(idx1, y1, idx2, y2, idx0, y0, n_dst):
    e_pads = tuple(_round_up(i.shape[1], 2 * _NS * _BS)
                   for i in (idx1, idx2, idx0))
    sa, da = _pad_edges(idx1, e_pads[0])
    sb, db = _pad_edges(idx2, e_pads[1])
    sc_, dc = _pad_edges(idx0, e_pads[2])
    R = _round_up(-(-n_dst // _C), 8)
    stripe = _round_up(R + 1, 128) // _NS
    z_h = jnp.zeros((stripe, _D), jnp.float32)
    return _make_spmm3(n_dst, e_pads)(y1, sa, da, y2, sb, db, y0, sc_, dc,
                                      z_h)


_BM = 2000


def _mm_body(x_ref, w_ref, o_ref):
    o_ref[...] = jnp.dot(x_ref[...], w_ref[...],
                         preferred_element_type=jnp.float32)


def _matmul(x, w):
    n = x.shape[0]
    return pl.pallas_call(
        _mm_body,
        grid=(n // _BM,),
        in_specs=[pl.BlockSpec((_BM, _D), lambda i: (i, 0)),
                  pl.BlockSpec((_D, _D), lambda i: (0, 0))],
        out_specs=pl.BlockSpec((_BM, _D), lambda i: (i, 0)),
        out_shape=jax.ShapeDtypeStruct((n, _D), jnp.float32),
    )(x, w)


def _stage1_body(scale_ref, s1_ref, s2_ref, x1_ref, wa_ref, wb_ref, b1_ref,
                 h_ref, sums_ref):
    i = pl.program_id(0)
    xs = x1_ref[...] * scale_ref[0, 0]
    xu = xs + _elu(s1_ref[...])
    xc = xs + _elu(s2_ref[...])
    h = (jnp.dot(xu, wa_ref[...], preferred_element_type=jnp.float32)
         + jnp.dot(xc, wb_ref[...], preferred_element_type=jnp.float32)
         + b1_ref[...])
    h_ref[...] = h
    ps = jnp.sum(h, axis=0)[None, :]
    pq = jnp.sum(h * h, axis=0)[None, :]
    blk = jnp.concatenate([ps, pq, jnp.zeros((6, _D), jnp.float32)], axis=0)

    @pl.when(i == 0)
    def _():
        sums_ref[...] = blk

    @pl.when(i > 0)
    def _():
        sums_ref[...] = sums_ref[...] + blk


def _stage1(s1, s2, x1, wa, wb, b1, scale):
    n = x1.shape[0]
    return pl.pallas_call(
        _stage1_body,
        grid=(n // _BM,),
        in_specs=[pl.BlockSpec(memory_space=pltpu.SMEM),
                  pl.BlockSpec((_BM, _D), lambda i: (i, 0)),
                  pl.BlockSpec((_BM, _D), lambda i: (i, 0)),
                  pl.BlockSpec((_BM, _D), lambda i: (i, 0)),
                  pl.BlockSpec((_D, _D), lambda i: (0, 0)),
                  pl.BlockSpec((_D, _D), lambda i: (0, 0)),
                  pl.BlockSpec((1, _D), lambda i: (0, 0))],
        out_specs=[pl.BlockSpec((_BM, _D), lambda i: (i, 0)),
                   pl.BlockSpec((8, _D), lambda i: (0, 0))],
        out_shape=[jax.ShapeDtypeStruct((n, _D), jnp.float32),
                   jax.ShapeDtypeStruct((8, _D), jnp.float32)],
    )(scale, s1, s2, x1, wa, wb, b1)


def _stage2_body(h_ref, s0_ref, sums_ref, g_ref, bt_ref, w2_ref, b2_ref,
                 wu_ref, bu_ref, o_ref, *, inv_n):
    mu = sums_ref[0:1, :] * inv_n
    var = sums_ref[1:2, :] * inv_n - mu * mu
    inv = lax.rsqrt(var + 1e-5) * g_ref[...]
    hn = jax.nn.relu((h_ref[...] - mu) * inv + bt_ref[...])
    xf = jnp.dot(hn, w2_ref[...], preferred_element_type=jnp.float32) + b2_ref[...]
    xa = xf + _elu(s0_ref[...])
    o_ref[...] = _elu(
        jnp.dot(xa, wu_ref[...], preferred_element_type=jnp.float32) + bu_ref[...])


def _stage2(h, s0, sums, gamma, beta, w2, b2, wu, bu):
    n = h.shape[0]
    return pl.pallas_call(
        functools.partial(_stage2_body, inv_n=1.0 / n),
        grid=(n // _BM,),
        in_specs=[pl.BlockSpec((_BM, _D), lambda i: (i, 0)),
                  pl.BlockSpec((_BM, _D), lambda i: (i, 0)),
                  pl.BlockSpec((8, _D), lambda i: (0, 0)),
                  pl.BlockSpec((1, _D), lambda i: (0, 0)),
                  pl.BlockSpec((1, _D), lambda i: (0, 0)),
                  pl.BlockSpec((_D, _D), lambda i: (0, 0)),
                  pl.BlockSpec((1, _D), lambda i: (0, 0)),
                  pl.BlockSpec((_D, _D), lambda i: (0, 0)),
                  pl.BlockSpec((1, _D), lambda i: (0, 0))],
        out_specs=pl.BlockSpec((_BM, _D), lambda i: (i, 0)),
        out_shape=jax.ShapeDtypeStruct((n, _D), jnp.float32),
    )(h, s0, sums, gamma, beta, w2, b2, wu, bu)


def kernel(x_0, x_1, x_2, neighborhood_1_to_1, neighborhood_2_to_1,
           neighborhood_0_to_1, W_1to1, W_2to1, mlp1_W1, mlp1_b1,
           mlp1_gamma, mlp1_beta, mlp1_W2, mlp1_b2, eps1, W_0to1,
           upd_W, upd_b):
    n1 = x_1.shape[0]
    y1 = _matmul(x_1, W_1to1)
    y2 = _matmul(x_2, W_2to1)
    y0 = _matmul(x_0, W_0to1)
    s1, s2, s0 = _spmm3(neighborhood_1_to_1, y1, neighborhood_2_to_1, y2,
                        neighborhood_0_to_1, y0, n1)
    scale = (1.0 + eps1).reshape(1, 1)
    wa = mlp1_W1[:_D]
    wb = mlp1_W1[_D:]
    h, sums = _stage1(s1, s2, x_1, wa, wb, mlp1_b1.reshape(1, _D), scale)
    return _stage2(h, s0, sums, mlp1_gamma.reshape(1, _D),
                   mlp1_beta.reshape(1, _D), mlp1_W2,
                   mlp1_b2.reshape(1, _D), upd_W, upd_b.reshape(1, _D))


# speedup vs baseline: 1.0066x; 1.0066x over previous
import functools

import jax
import jax.numpy as jnp
from jax import lax
from jax.experimental import pallas as pl
from jax.experimental.pallas import tpu as pltpu
from jax.experimental.pallas import tpu_sc as plsc

_NC = 2
_NS = 16
_LN = 16
_D = 128
_DSUB = _D // _LN
_KPC = _DSUB // _NC
_K = 512


def _round_up(x, m):
    return (x + m - 1) // m * m


def _elu(x):
    return jnp.where(x > 0, x, jnp.exp(jnp.minimum(x, 0.0)) - 1.0)


_C = 12
_NU = 3
_UR = 128
_GU = _NU * _UR
_BS = 2048
_CAP = 2 * _BS + _GU + 16
_RP = 16384


@functools.cache
def _make_spmm3(n_dst, e_pads):
    ets = [e // _NS for e in e_pads]
    nblks = [et // _BS for et in ets]
    assert all(nb * _BS == et for nb, et in zip(nblks, ets))
    R = _round_up(-(-n_dst // _C), 8)
    assert R < _RP
    acc_rows = _round_up(R + 1, 128)
    stripe = acc_rows // _NS
    trash = R
    dtail = R - (_NS - 1) * stripe
    assert 0 < dtail <= stripe and dtail % 8 == 0
    mesh = plsc.VectorSubcoreMesh(core_axis_name="c", subcore_axis_name="s",
                                  num_cores=_NC, num_subcores=_NS)
    out = jax.ShapeDtypeStruct((_C * R, _D), jnp.float32)

    scratch = [
        pltpu.VMEM((_BS,), jnp.int32),
        pltpu.VMEM((_BS,), jnp.int32),
        pltpu.VMEM((_BS,), jnp.int32),
        pltpu.VMEM((_BS,), jnp.int32),
        pltpu.VMEM((_CAP,), jnp.int32),
    ]
    scratch += [pltpu.VMEM((_UR,), jnp.int32) for _ in range(2 * _NU)]
    scratch += [pltpu.VMEM((_UR, _D), jnp.float32) for _ in range(_NU)]
    scratch += [pltpu.VMEM_SHARED((acc_rows, _D), jnp.float32)]
    scratch += [pltpu.SemaphoreType.DMA for _ in range(2 + _NU)]

    @functools.partial(
        pl.kernel,
        out_type=(out, out, out),
        mesh=mesh,
        compiler_params=pltpu.CompilerParams(needs_layout_passes=False),
        scratch_types=scratch,
    )
    def spmm3(ya, sa, da, yb, sb, db, yc, sc_, dc, z_h, outa, outb, outc,
              s0, d0, s1, d1, cbuf, *rest):
        gb = rest[:_NU]
        ib = rest[_NU:2 * _NU]
        rb = rest[2 * _NU:3 * _NU]
        acc = rest[3 * _NU]
        semA, semB = rest[3 * _NU + 1], rest[3 * _NU + 2]
        gsem = rest[3 * _NU + 3:]
        c = lax.axis_index("c")
        s = lax.axis_index("s")
        ti = jnp.full((_LN,), trash, jnp.int32)
        ones = jnp.ones((_LN,), jnp.int32)

        r0 = s * stripe
        stage = ((s0, d0, semA), (s1, d1, semB))
        sets = ((ya, sa, da, outa, 0), (yb, sb, db, outb, 1),
                (yc, sc_, dc, outc, 2))
        for y_h, src_h, dst_h, out_h, si in sets:
            et = ets[si]
            nblk = nblks[si]
            e0 = s * et

            def do_chunk(cc, _, y_h=y_h, src_h=src_h, dst_h=dst_h,
                         out_h=out_h, et=et, nblk=nblk, e0=e0):
                ch = c * (_C // _NC) + cc
                lo = ch * R
                hi = lo + R

                pltpu.sync_copy(z_h, acc.at[pl.ds(r0, stripe)])
                plsc.subcore_barrier()

                def flushg(g, _):
                    base = g * _GU
                    cps = []
                    for u in range(_NU):
                        def fill(j, _, u=u):
                            cv = cbuf[pl.ds(base + u * _UR + j * _LN, _LN)]
                            gb[u][pl.ds(j * _LN, _LN)] = cv >> 14
                            ib[u][pl.ds(j * _LN, _LN)] = cv & (_RP - 1)
                            return 0
                        lax.fori_loop(0, _UR // _LN, fill, 0)
                        cp = pltpu.make_async_copy(y_h.at[gb[u]], rb[u],
                                                   gsem[u])
                        cp.start()
                        cps.append(cp)
                    for u in range(_NU):
                        cps[u].wait()
                        pltpu.sync_copy(rb[u], acc.at[ib[u]], add=True)
                    return 0

                def scan_vregs(buf_s, buf_d, n):
                    def scan_v(i, n):
                        d = buf_d[pl.ds(i * _LN, _LN)]
                        sv = buf_s[pl.ds(i * _LN, _LN)]
                        m = (d >= lo) & (d < hi)
                        pos = plsc.cumsum(ones, mask=m) + (n - 1)
                        plsc.store_scatter(cbuf, [pos],
                                           sv * _RP + (d - lo), mask=m)
                        return n + plsc.all_reduce_population_count(m)[0]
                    return lax.fori_loop(0, _BS // _LN, scan_v, n)

                def flush_compact(n):
                    ng = n // _GU
                    lax.fori_loop(0, ng, flushg, 0)
                    done = ng * _GU

                    def comp(j, _):
                        cbuf[pl.ds(j * _LN, _LN)] = cbuf[
                            pl.ds(done + j * _LN, _LN)]
                        return 0
                    lax.fori_loop(0, _GU // _LN, comp, 0)
                    return n - done

                def stage_pair(p):
                    eb0 = e0 + (2 * p) * _BS
                    eb1 = eb0 + _BS
                    pltpu.make_async_copy(src_h.at[pl.ds(eb0, _BS)], s0,
                                          semA).start()
                    pltpu.make_async_copy(dst_h.at[pl.ds(eb0, _BS)], d0,
                                          semA).start()
                    pltpu.make_async_copy(src_h.at[pl.ds(eb1, _BS)], s1,
                                          semB).start()
                    pltpu.make_async_copy(dst_h.at[pl.ds(eb1, _BS)], d1,
                                          semB).start()

                def wait_stage(buf, sem):
                    pltpu.make_async_copy(src_h.at[pl.ds(e0, _BS)], buf,
                                          sem).wait()

                stage_pair(0)

                def pair(p, n):
                    wait_stage(s0, semA)
                    wait_stage(d0, semA)
                    n = scan_vregs(s0, d0, n)
                    wait_stage(s1, semB)
                    wait_stage(d1, semB)
                    n = scan_vregs(s1, d1, n)

                    @pl.when(p + 1 < nblk // 2)
                    def _():
                        stage_pair(p + 1)
                    return flush_compact(n)
                n = lax.fori_loop(0, nblk // 2, pair, 0)

                def padf(j, _):
                    cbuf[pl.ds(n + j * _LN, _LN)] = ti
                    return 0
                lax.fori_loop(0, _GU // _LN, padf, 0)
                lax.fori_loop(0, (n + _GU - 1) // _GU, flushg, 0)
                plsc.subcore_barrier()

                @pl.when(s < _NS - 1)
                def _():
                    pltpu.sync_copy(acc.at[pl.ds(r0, stripe)],
                                    out_h.at[pl.ds(lo + r0, stripe)])

                @pl.when(s == _NS - 1)
                def _():
                    pltpu.sync_copy(acc.at[pl.ds(r0, dtail)],
                                    out_h.at[pl.ds(lo + r0, dtail)])
                plsc.subcore_barrier()
                return 0
            lax.fori_loop(0, _C // _NC, do_chunk, 0)

    return spmm3


def _pad_edges(idx, e_pad):
    pad = e_pad - idx.shape[1]
    dst_p = jnp.concatenate([idx[0], jnp.full((pad,), 1 << 30, jnp.int32)])
    src_p = jnp.concatenate([idx[1], jnp.zeros((pad,), jnp.int32)])
    return src_p, dst_p


def _spmm3(idx1, y1, idx2, y2, idx0, y0, n_dst):
    e_pads = tuple(_round_up(i.shape[1], 2 * _NS * _BS)
                   for i in (idx1, idx2, idx0))
    sa, da = _pad_edges(idx1, e_pads[0])
    sb, db = _pad_edges(idx2, e_pads[1])
    sc_, dc = _pad_edges(idx0, e_pads[2])
    R = _round_up(-(-n_dst // _C), 8)
    stripe = _round_up(R + 1, 128) // _NS
    z_h = jnp.zeros((stripe, _D), jnp.float32)
    return _make_spmm3(n_dst, e_pads)(y1, sa, da, y2, sb, db, y0, sc_, dc,
                                      z_h)


_BM = 2000


def _mm_body(x_ref, w_ref, o_ref):
    o_ref[...] = jnp.dot(x_ref[...], w_ref[...],
                         preferred_element_type=jnp.float32)


def _matmul(x, w):
    n = x.shape[0]
    return pl.pallas_call(
        _mm_body,
        grid=(n // _BM,),
        in_specs=[pl.BlockSpec((_BM, _D), lambda i: (i, 0)),
                  pl.BlockSpec((_D, _D), lambda i: (0, 0))],
        out_specs=pl.BlockSpec((_BM, _D), lambda i: (i, 0)),
        out_shape=jax.ShapeDtypeStruct((n, _D), jnp.float32),
    )(x, w)


def _stage1_body(scale_ref, s1_ref, s2_ref, x1_ref, wa_ref, wb_ref, b1_ref,
                 h_ref, sums_ref):
    i = pl.program_id(0)
    xs = x1_ref[...] * scale_ref[0, 0]
    xu = xs + _elu(s1_ref[...])
    xc = xs + _elu(s2_ref[...])
    h = (jnp.dot(xu, wa_ref[...], preferred_element_type=jnp.float32)
         + jnp.dot(xc, wb_ref[...], preferred_element_type=jnp.float32)
         + b1_ref[...])
    h_ref[...] = h
    ps = jnp.sum(h, axis=0)[None, :]
    pq = jnp.sum(h * h, axis=0)[None, :]
    blk = jnp.concatenate([ps, pq, jnp.zeros((6, _D), jnp.float32)], axis=0)

    @pl.when(i == 0)
    def _():
        sums_ref[...] = blk

    @pl.when(i > 0)
    def _():
        sums_ref[...] = sums_ref[...] + blk


def _stage1(s1, s2, x1, wa, wb, b1, scale):
    n = x1.shape[0]
    return pl.pallas_call(
        _stage1_body,
        grid=(n // _BM,),
        in_specs=[pl.BlockSpec(memory_space=pltpu.SMEM),
                  pl.BlockSpec((_BM, _D), lambda i: (i, 0)),
                  pl.BlockSpec((_BM, _D), lambda i: (i, 0)),
                  pl.BlockSpec((_BM, _D), lambda i: (i, 0)),
                  pl.BlockSpec((_D, _D), lambda i: (0, 0)),
                  pl.BlockSpec((_D, _D), lambda i: (0, 0)),
                  pl.BlockSpec((1, _D), lambda i: (0, 0))],
        out_specs=[pl.BlockSpec((_BM, _D), lambda i: (i, 0)),
                   pl.BlockSpec((8, _D), lambda i: (0, 0))],
        out_shape=[jax.ShapeDtypeStruct((n, _D), jnp.float32),
                   jax.ShapeDtypeStruct((8, _D), jnp.float32)],
    )(scale, s1, s2, x1, wa, wb, b1)


def _stage2_body(h_ref, s0_ref, sums_ref, g_ref, bt_ref, w2_ref, b2_ref,
                 wu_ref, bu_ref, o_ref, *, inv_n):
    mu = sums_ref[0:1, :] * inv_n
    var = sums_ref[1:2, :] * inv_n - mu * mu
    inv = lax.rsqrt(var + 1e-5) * g_ref[...]
    hn = jax.nn.relu((h_ref[...] - mu) * inv + bt_ref[...])
    xf = jnp.dot(hn, w2_ref[...], preferred_element_type=jnp.float32) + b2_ref[...]
    xa = xf + _elu(s0_ref[...])
    o_ref[...] = _elu(
        jnp.dot(xa, wu_ref[...], preferred_element_type=jnp.float32) + bu_ref[...])


def _stage2(h, s0, sums, gamma, beta, w2, b2, wu, bu):
    n = h.shape[0]
    return pl.pallas_call(
        functools.partial(_stage2_body, inv_n=1.0 / n),
        grid=(n // _BM,),
        in_specs=[pl.BlockSpec((_BM, _D), lambda i: (i, 0)),
                  pl.BlockSpec((_BM, _D), lambda i: (i, 0)),
                  pl.BlockSpec((8, _D), lambda i: (0, 0)),
                  pl.BlockSpec((1, _D), lambda i: (0, 0)),
                  pl.BlockSpec((1, _D), lambda i: (0, 0)),
                  pl.BlockSpec((_D, _D), lambda i: (0, 0)),
                  pl.BlockSpec((1, _D), lambda i: (0, 0)),
                  pl.BlockSpec((_D, _D), lambda i: (0, 0)),
                  pl.BlockSpec((1, _D), lambda i: (0, 0))],
        out_specs=pl.BlockSpec((_BM, _D), lambda i: (i, 0)),
        out_shape=jax.ShapeDtypeStruct((n, _D), jnp.float32),
    )(h, s0, sums, gamma, beta, w2, b2, wu, bu)


def kernel(x_0, x_1, x_2, neighborhood_1_to_1, neighborhood_2_to_1,
           neighborhood_0_to_1, W_1to1, W_2to1, mlp1_W1, mlp1_b1,
           mlp1_gamma, mlp1_beta, mlp1_W2, mlp1_b2, eps1, W_0to1,
           upd_W, upd_b):
    n1 = x_1.shape[0]
    y1 = _matmul(x_1, W_1to1)
    y2 = _matmul(x_2, W_2to1)
    y0 = _matmul(x_0, W_0to1)
    s1, s2, s0 = _spmm3(neighborhood_1_to_1, y1, neighborhood_2_to_1, y2,
                        neighborhood_0_to_1, y0, n1)
    scale = (1.0 + eps1).reshape(1, 1)
    wa = mlp1_W1[:_D]
    wb = mlp1_W1[_D:]
    h, sums = _stage1(s1, s2, x_1, wa, wb, mlp1_b1.reshape(1, _D), scale)
    return _stage2(h, s0, sums, mlp1_gamma.reshape(1, _D),
                   mlp1_beta.reshape(1, _D), mlp1_W2,
                   mlp1_b2.reshape(1, _D), upd_W, upd_b.reshape(1, _D))


# speedup vs baseline: 1.7550x; 1.7436x over previous
import functools

import jax
import jax.numpy as jnp
from jax import lax
from jax.experimental import pallas as pl
from jax.experimental.pallas import tpu as pltpu
from jax.experimental.pallas import tpu_sc as plsc

_NC = 2
_NS = 16
_LN = 16
_D = 128
_DSUB = _D // _LN
_KPC = _DSUB // _NC
_K = 512


def _round_up(x, m):
    return (x + m - 1) // m * m


def _elu(x):
    return jnp.where(x > 0, x, jnp.exp(jnp.minimum(x, 0.0)) - 1.0)


_C = 12
_NU = 2
_UR = 128
_GU = _NU * _UR
_BS = 2048
_CAP = 2 * _BS + _GU + 16
_RP = 16384


@functools.cache
def _make_spmm3(n_dst, e_pads):
    ets = [e // _NS for e in e_pads]
    nblks = [et // _BS for et in ets]
    assert all(nb * _BS == et for nb, et in zip(nblks, ets))
    R = _round_up(-(-n_dst // _C), 8)
    assert R < _RP
    acc_rows = _round_up(R + 1, 128)
    stripe = acc_rows // _NS
    trash = R
    dtail = R - (_NS - 1) * stripe
    assert 0 < dtail <= stripe and dtail % 8 == 0
    mesh = plsc.VectorSubcoreMesh(core_axis_name="c", subcore_axis_name="s",
                                  num_cores=_NC, num_subcores=_NS)
    out = jax.ShapeDtypeStruct((_C * R, _D), jnp.float32)

    scratch = [
        pltpu.VMEM((_BS,), jnp.int32),
        pltpu.VMEM((_BS,), jnp.int32),
        pltpu.VMEM((_BS,), jnp.int32),
        pltpu.VMEM((_BS,), jnp.int32),
        pltpu.VMEM((_CAP,), jnp.int32),
    ]
    scratch += [pltpu.VMEM((_UR,), jnp.int32) for _ in range(2 * _NU)]
    scratch += [pltpu.VMEM((_UR, _D), jnp.float32) for _ in range(_NU)]
    scratch += [pltpu.VMEM_SHARED((acc_rows, _D), jnp.float32)]
    scratch += [pltpu.SemaphoreType.DMA for _ in range(2 + _NU)]

    @functools.partial(
        pl.kernel,
        out_type=(out, out, out),
        mesh=mesh,
        compiler_params=pltpu.CompilerParams(needs_layout_passes=False),
        scratch_types=scratch,
    )
    def spmm3(ya, sa, da, yb, sb, db, yc, sc_, dc, z_h, outa, outb, outc,
              s0, d0, s1, d1, cbuf, *rest):
        gb = rest[:_NU]
        ib = rest[_NU:2 * _NU]
        rb = rest[2 * _NU:3 * _NU]
        acc = rest[3 * _NU]
        semA, semB = rest[3 * _NU + 1], rest[3 * _NU + 2]
        gsem = rest[3 * _NU + 3:]
        c = lax.axis_index("c")
        s = lax.axis_index("s")
        ti = jnp.full((_LN,), trash, jnp.int32)
        ones = jnp.ones((_LN,), jnp.int32)

        r0 = s * stripe
        stage = ((s0, d0, semA), (s1, d1, semB))
        sets = ((ya, sa, da, outa, 0), (yb, sb, db, outb, 1),
                (yc, sc_, dc, outc, 2))
        for y_h, src_h, dst_h, out_h, si in sets:
            et = ets[si]
            nblk = nblks[si]
            e0 = s * et

            def do_chunk(cc, _, y_h=y_h, src_h=src_h, dst_h=dst_h,
                         out_h=out_h, et=et, nblk=nblk, e0=e0):
                ch = c * (_C // _NC) + cc
                lo = ch * R
                hi = lo + R

                pltpu.sync_copy(z_h, acc.at[pl.ds(r0, stripe)])
                plsc.subcore_barrier()

                def flushg(g, _):
                    base = g * _GU
                    cps = []
                    for u in range(_NU):
                        def fill(j, _, u=u):
                            cv = cbuf[pl.ds(base + u * _UR + j * _LN, _LN)]
                            gb[u][pl.ds(j * _LN, _LN)] = cv >> 14
                            ib[u][pl.ds(j * _LN, _LN)] = cv & (_RP - 1)
                            return 0
                        lax.fori_loop(0, _UR // _LN, fill, 0)
                        cp = pltpu.make_async_copy(y_h.at[gb[u]], rb[u],
                                                   gsem[u])
                        cp.start()
                        cps.append(cp)
                    for u in range(_NU):
                        cps[u].wait()
                        pltpu.sync_copy(rb[u], acc.at[ib[u]], add=True)
                    return 0

                def scan_vregs(buf_s, buf_d, n):
                    def scan_v(i, n):
                        d = buf_d[pl.ds(i * _LN, _LN)]
                        sv = buf_s[pl.ds(i * _LN, _LN)]
                        m = (d >= lo) & (d < hi)
                        pos = plsc.cumsum(ones, mask=m) + (n - 1)
                        plsc.store_scatter(cbuf, [pos],
                                           sv * _RP + (d - lo), mask=m)
                        return n + plsc.all_reduce_population_count(m)[0]
                    return lax.fori_loop(0, _BS // _LN, scan_v, n)

                def flush_compact(n):
                    ng = n // _GU
                    lax.fori_loop(0, ng, flushg, 0)
                    done = ng * _GU

                    def comp(j, _):
                        cbuf[pl.ds(j * _LN, _LN)] = cbuf[
                            pl.ds(done + j * _LN, _LN)]
                        return 0
                    lax.fori_loop(0, _GU // _LN, comp, 0)
                    return n - done

                def stage_pair(p):
                    eb0 = e0 + (2 * p) * _BS
                    eb1 = eb0 + _BS
                    pltpu.make_async_copy(src_h.at[pl.ds(eb0, _BS)], s0,
                                          semA).start()
                    pltpu.make_async_copy(dst_h.at[pl.ds(eb0, _BS)], d0,
                                          semA).start()
                    pltpu.make_async_copy(src_h.at[pl.ds(eb1, _BS)], s1,
                                          semB).start()
                    pltpu.make_async_copy(dst_h.at[pl.ds(eb1, _BS)], d1,
                                          semB).start()

                def wait_stage(buf, sem):
                    pltpu.make_async_copy(src_h.at[pl.ds(e0, _BS)], buf,
                                          sem).wait()

                stage_pair(0)

                def pair(p, n):
                    wait_stage(s0, semA)
                    wait_stage(d0, semA)
                    n = scan_vregs(s0, d0, n)
                    wait_stage(s1, semB)
                    wait_stage(d1, semB)
                    n = scan_vregs(s1, d1, n)

                    @pl.when(p + 1 < nblk // 2)
                    def _():
                        stage_pair(p + 1)
                    return flush_compact(n)
                n = lax.fori_loop(0, nblk // 2, pair, 0)

                def padf(j, _):
                    cbuf[pl.ds(n + j * _LN, _LN)] = ti
                    return 0
                lax.fori_loop(0, _GU // _LN, padf, 0)
                lax.fori_loop(0, (n + _GU - 1) // _GU, flushg, 0)
                plsc.subcore_barrier()

                @pl.when(s < _NS - 1)
                def _():
                    pltpu.sync_copy(acc.at[pl.ds(r0, stripe)],
                                    out_h.at[pl.ds(lo + r0, stripe)])

                @pl.when(s == _NS - 1)
                def _():
                    pltpu.sync_copy(acc.at[pl.ds(r0, dtail)],
                                    out_h.at[pl.ds(lo + r0, dtail)])
                plsc.subcore_barrier()
                return 0
            lax.fori_loop(0, _C // _NC, do_chunk, 0)

    return spmm3


def _pad_edges(idx, e_pad):
    pad = e_pad - idx.shape[1]
    dst_p = jnp.concatenate([idx[0], jnp.full((pad,), 1 << 30, jnp.int32)])
    src_p = jnp.concatenate([idx[1], jnp.zeros((pad,), jnp.int32)])
    return src_p, dst_p


def _spmm3(idx1, y1, idx2, y2, idx0, y0, n_dst):
    e_pads = tuple(_round_up(i.shape[1], 2 * _NS * _BS)
                   for i in (idx1, idx2, idx0))
    sa, da = _pad_edges(idx1, e_pads[0])
    sb, db = _pad_edges(idx2, e_pads[1])
    sc_, dc = _pad_edges(idx0, e_pads[2])
    R = _round_up(-(-n_dst // _C), 8)
    stripe = _round_up(R + 1, 128) // _NS
    z_h = jnp.zeros((stripe, _D), jnp.float32)
    return _make_spmm3(n_dst, e_pads)(y1, sa, da, y2, sb, db, y0, sc_, dc,
                                      z_h)


_BM = 2000


def _mm_body(x_ref, w_ref, o_ref):
    o_ref[...] = jnp.dot(x_ref[...], w_ref[...],
                         preferred_element_type=jnp.float32)


def _matmul(x, w):
    n = x.shape[0]
    return pl.pallas_call(
        _mm_body,
        grid=(n // _BM,),
        in_specs=[pl.BlockSpec((_BM, _D), lambda i: (i, 0)),
                  pl.BlockSpec((_D, _D), lambda i: (0, 0))],
        out_specs=pl.BlockSpec((_BM, _D), lambda i: (i, 0)),
        out_shape=jax.ShapeDtypeStruct((n, _D), jnp.float32),
    )(x, w)


def _stage1_body(scale_ref, s1_ref, s2_ref, x1_ref, wa_ref, wb_ref, b1_ref,
                 h_ref, sums_ref):
    i = pl.program_id(0)
    xs = x1_ref[...] * scale_ref[0, 0]
    xu = xs + _elu(s1_ref[...])
    xc = xs + _elu(s2_ref[...])
    h = (jnp.dot(xu, wa_ref[...], preferred_element_type=jnp.float32)
         + jnp.dot(xc, wb_ref[...], preferred_element_type=jnp.float32)
         + b1_ref[...])
    h_ref[...] = h
    ps = jnp.sum(h, axis=0)[None, :]
    pq = jnp.sum(h * h, axis=0)[None, :]
    blk = jnp.concatenate([ps, pq, jnp.zeros((6, _D), jnp.float32)], axis=0)

    @pl.when(i == 0)
    def _():
        sums_ref[...] = blk

    @pl.when(i > 0)
    def _():
        sums_ref[...] = sums_ref[...] + blk


def _stage1(s1, s2, x1, wa, wb, b1, scale):
    n = x1.shape[0]
    return pl.pallas_call(
        _stage1_body,
        grid=(n // _BM,),
        in_specs=[pl.BlockSpec(memory_space=pltpu.SMEM),
                  pl.BlockSpec((_BM, _D), lambda i: (i, 0)),
                  pl.BlockSpec((_BM, _D), lambda i: (i, 0)),
                  pl.BlockSpec((_BM, _D), lambda i: (i, 0)),
                  pl.BlockSpec((_D, _D), lambda i: (0, 0)),
                  pl.BlockSpec((_D, _D), lambda i: (0, 0)),
                  pl.BlockSpec((1, _D), lambda i: (0, 0))],
        out_specs=[pl.BlockSpec((_BM, _D), lambda i: (i, 0)),
                   pl.BlockSpec((8, _D), lambda i: (0, 0))],
        out_shape=[jax.ShapeDtypeStruct((n, _D), jnp.float32),
                   jax.ShapeDtypeStruct((8, _D), jnp.float32)],
    )(scale, s1, s2, x1, wa, wb, b1)


def _stage2_body(h_ref, s0_ref, sums_ref, g_ref, bt_ref, w2_ref, b2_ref,
                 wu_ref, bu_ref, o_ref, *, inv_n):
    mu = sums_ref[0:1, :] * inv_n
    var = sums_ref[1:2, :] * inv_n - mu * mu
    inv = lax.rsqrt(var + 1e-5) * g_ref[...]
    hn = jax.nn.relu((h_ref[...] - mu) * inv + bt_ref[...])
    xf = jnp.dot(hn, w2_ref[...], preferred_element_type=jnp.float32) + b2_ref[...]
    xa = xf + _elu(s0_ref[...])
    o_ref[...] = _elu(
        jnp.dot(xa, wu_ref[...], preferred_element_type=jnp.float32) + bu_ref[...])


def _stage2(h, s0, sums, gamma, beta, w2, b2, wu, bu):
    n = h.shape[0]
    return pl.pallas_call(
        functools.partial(_stage2_body, inv_n=1.0 / n),
        grid=(n // _BM,),
        in_specs=[pl.BlockSpec((_BM, _D), lambda i: (i, 0)),
                  pl.BlockSpec((_BM, _D), lambda i: (i, 0)),
                  pl.BlockSpec((8, _D), lambda i: (0, 0)),
                  pl.BlockSpec((1, _D), lambda i: (0, 0)),
                  pl.BlockSpec((1, _D), lambda i: (0, 0)),
                  pl.BlockSpec((_D, _D), lambda i: (0, 0)),
                  pl.BlockSpec((1, _D), lambda i: (0, 0)),
                  pl.BlockSpec((_D, _D), lambda i: (0, 0)),
                  pl.BlockSpec((1, _D), lambda i: (0, 0))],
        out_specs=pl.BlockSpec((_BM, _D), lambda i: (i, 0)),
        out_shape=jax.ShapeDtypeStruct((n, _D), jnp.float32),
    )(h, s0, sums, gamma, beta, w2, b2, wu, bu)


def kernel(x_0, x_1, x_2, neighborhood_1_to_1, neighborhood_2_to_1,
           neighborhood_0_to_1, W_1to1, W_2to1, mlp1_W1, mlp1_b1,
           mlp1_gamma, mlp1_beta, mlp1_W2, mlp1_b2, eps1, W_0to1,
           upd_W, upd_b):
    n1 = x_1.shape[0]
    y1 = _matmul(x_1, W_1to1)
    y2 = _matmul(x_2, W_2to1)
    y0 = _matmul(x_0, W_0to1)
    s1, s2, s0 = _spmm3(neighborhood_1_to_1, y1, neighborhood_2_to_1, y2,
                        neighborhood_0_to_1, y0, n1)
    scale = (1.0 + eps1).reshape(1, 1)
    wa = mlp1_W1[:_D]
    wb = mlp1_W1[_D:]
    h, sums = _stage1(s1, s2, x_1, wa, wb, mlp1_b1.reshape(1, _D), scale)
    return _stage2(h, s0, sums, mlp1_gamma.reshape(1, _D),
                   mlp1_beta.reshape(1, _D), mlp1_W2,
                   mlp1_b2.reshape(1, _D), upd_W, upd_b.reshape(1, _D))
